# baseline (speedup 1.0000x reference)
import functools

import jax
import jax.numpy as jnp
from jax import lax
from jax.experimental import pallas as pl
from jax.experimental.pallas import tpu as pltpu
from jax.experimental.pallas import tpu_sc as plsc

_N = 50000
_MP = 25000
_K = 16
_C = 128
_L = 16
_NW = 32
_P = 784
_PCUT = 21
_MP_PAD = _NW * _P
_VMAX = _P * _K + _K
_CH = 128
_RING = 4
_RROWS = _RING * _CH
_OB = 8


def _pool_body(inp_hbm, cnt_hbm, idx_hbm, out_hbm,
               idx_vf, cnt_v, cidx, ring, stage, sems_g, sems_o):
    wid = lax.axis_index("s") * 2 + lax.axis_index("c")
    pw = jnp.where(wid < _PCUT, _P, _P - 8)
    out_base = wid * _P - jnp.maximum(wid - _PCUT, 0) * 8
    sbase = jnp.minimum(out_base, _MP - _P)
    loc = out_base - sbase

    pltpu.sync_copy(idx_hbm.at[pl.ds(sbase * _K, _P * _K)],
                    idx_vf.at[pl.ds(0, _P * _K)])
    pltpu.sync_copy(cnt_hbm.at[pl.ds(sbase, _P)], cnt_v.at[pl.ds(0, _P)])

    def compact(p, off):
        row = idx_vf[pl.ds((p + loc) * _K, _K)]
        cidx[pl.ds(off, _K)] = row
        cnt = cnt_v[pl.ds(p + loc, _L)][0]
        return off + cnt

    total = lax.fori_loop(0, pw, compact, jnp.int32(0))
    zeros = jnp.zeros((_L,), jnp.int32)
    for t in range(_CH // _L):
        cidx[pl.ds(total + t * _L, _L)] = zeros
    nch = (total + _CH - 1) >> 7

    def gather(chunk):
        slot = chunk & (_RING - 1)
        dst0 = pl.multiple_of(slot * _CH, _CH)
        pltpu.async_copy(
            inp_hbm.at[cidx.at[pl.ds(chunk * _CH, _CH)]],
            ring.at[pl.ds(dst0, _CH)],
            sems_g.at[slot])

    def wait_gather(slot):
        pltpu.make_async_copy(
            inp_hbm.at[pl.ds(0, _CH)], ring.at[pl.ds(0, _CH)],
            sems_g.at[slot]).wait()

    def wait_flush(slot):
        pltpu.make_async_copy(
            stage.at[pl.ds(0, _OB)], out_hbm.at[pl.ds(0, _OB)],
            sems_o.at[slot]).wait()

    for c in range(_RING):
        gather(jnp.int32(c))

    sls = [pl.ds(c * _L, _L) for c in range(_C // _L)]

    def point(p, carry):
        off, gathered, issued, srow = carry
        cnt = cnt_v[pl.ds(p + loc, _L)][0]
        last_chunk = (off + cnt - 1) >> 7

        @pl.when(last_chunk >= gathered)
        def _():
            wait_gather(gathered & (_RING - 1))

        gathered = jnp.where(last_chunk >= gathered, gathered + 1, gathered)

        can_issue = (issued < nch) & ((issued - _RING) < (off >> 7))

        @pl.when(can_issue)
        def _():
            gather(issued)

        issued = jnp.where(can_issue, issued + 1, issued)

        half = srow >> 3

        @pl.when(((srow == 0) | (srow == _OB)) & (p >= 2 * _OB))
        def _():
            wait_flush(half)

        r0 = off & (_RROWS - 1)
        acc = tuple(ring[r0, sl] for sl in sls)

        def fold(j, acc):
            ra = (off + 1 + 2 * j) & (_RROWS - 1)
            rb = (off + 2 + 2 * j) & (_RROWS - 1)
            return tuple(
                jnp.maximum(jnp.maximum(a, ring[ra, sl]), ring[rb, sl])
                for a, sl in zip(acc, sls))

        acc = lax.fori_loop(0, (cnt - 1) >> 1, fold, acc)
        rt = (off + jnp.where((cnt & 1) == 1, 0, cnt - 1)) & (_RROWS - 1)
        acc = tuple(jnp.maximum(a, ring[rt, sl])
                    for a, sl in zip(acc, sls))
        for a, sl in zip(acc, sls):
            stage[srow, sl] = a

        @pl.when((srow == _OB - 1) | (srow == 2 * _OB - 1))
        def _():
            s0 = pl.multiple_of((half << 3), _OB)
            d0 = pl.multiple_of(out_base + p - (_OB - 1), 8)
            pltpu.async_copy(stage.at[pl.ds(s0, _OB)],
                             out_hbm.at[pl.ds(d0, _OB)], sems_o.at[half])

        srow = jnp.where(srow == 2 * _OB - 1, 0, srow + 1)
        return off + cnt, gathered, issued, srow

    lax.fori_loop(0, pw, point,
                  (jnp.int32(0), jnp.int32(0), jnp.int32(_RING),
                   jnp.int32(0)))
    wait_flush(jnp.int32(0))
    wait_flush(jnp.int32(1))


_pool_call = functools.partial(
    pl.kernel,
    out_type=jax.ShapeDtypeStruct((_MP, _C), jnp.float32),
    mesh=plsc.VectorSubcoreMesh(core_axis_name="c", subcore_axis_name="s"),
    scratch_types=[
        pltpu.VMEM((_P * _K,), jnp.int32),
        pltpu.VMEM((_P + _L,), jnp.int32),
        pltpu.VMEM((_VMAX + _CH + _L,), jnp.int32),
        pltpu.VMEM((_RROWS, _C), jnp.float32),
        pltpu.VMEM((2 * _OB, _C), jnp.float32),
        pltpu.SemaphoreType.DMA((_RING,)),
        pltpu.SemaphoreType.DMA((2,)),
    ],
)(_pool_body)


def kernel(inputs, nn_count, nn_index):
    idx = nn_index.astype(jnp.int32)
    cnt = nn_count.astype(jnp.int32)
    return _pool_call(inputs, cnt, idx.reshape(-1))


# speedup vs baseline: 1.1455x; 1.1455x over previous
import functools

import jax
import jax.numpy as jnp
from jax import lax
from jax.experimental import pallas as pl
from jax.experimental.pallas import tpu as pltpu
from jax.experimental.pallas import tpu_sc as plsc

_N = 50000
_MP = 25000
_K = 16
_C = 128
_L = 16
_NW = 32
_P = 784
_PCUT = 21
_VMAX = _P * _K + _K
_CH = 128
_RING = 4
_RROWS = _RING * _CH
_OB = 8


def _pool_body(inp_hbm, cnt_hbm, idx_hbm, out_hbm,
               idx_vf, cnt_v, cidx, ring, stage, sems_g, sems_o):
    wid = lax.axis_index("s") * 2 + lax.axis_index("c")
    pw = jnp.where(wid < _PCUT, _P, _P - 8)
    out_base = wid * _P - jnp.maximum(wid - _PCUT, 0) * 8
    sbase = jnp.minimum(out_base, _MP - _P)
    loc = out_base - sbase

    pltpu.sync_copy(idx_hbm.at[pl.ds(sbase * _K, _P * _K)],
                    idx_vf.at[pl.ds(0, _P * _K)])
    pltpu.sync_copy(cnt_hbm.at[pl.ds(sbase, _P)], cnt_v.at[pl.ds(0, _P)])

    def compact(p, off):
        row = idx_vf[pl.ds((p + loc) * _K, _K)]
        cidx[pl.ds(off, _K)] = row
        cnt = cnt_v[pl.ds(p + loc, _L)][0]
        cidx[pl.ds(off + cnt, _L)] = jnp.full((_L,), row[0], jnp.int32)
        return off + cnt + (cnt & 1)

    total = lax.fori_loop(0, pw, compact, jnp.int32(0))
    zeros = jnp.zeros((_L,), jnp.int32)
    for t in range(_CH // _L):
        cidx[pl.ds(total + t * _L, _L)] = zeros
    nch = (total + _CH - 1) >> 7

    def gather(chunk):
        slot = chunk & (_RING - 1)
        dst0 = pl.multiple_of(slot * _CH, _CH)
        pltpu.async_copy(
            inp_hbm.at[cidx.at[pl.ds(chunk * _CH, _CH)]],
            ring.at[pl.ds(dst0, _CH)],
            sems_g.at[slot])

    def wait_gather(slot):
        pltpu.make_async_copy(
            inp_hbm.at[pl.ds(0, _CH)], ring.at[pl.ds(0, _CH)],
            sems_g.at[slot]).wait()

    def wait_flush(slot):
        pltpu.make_async_copy(
            stage.at[pl.ds(0, _OB)], out_hbm.at[pl.ds(0, _OB)],
            sems_o.at[slot]).wait()

    for c in range(_RING):
        gather(jnp.int32(c))

    sls = [pl.ds(c * _L, _L) for c in range(_C // _L)]

    def point(p, carry):
        off, gathered, issued, srow = carry
        cnt = cnt_v[pl.ds(p + loc, _L)][0]
        cnt_p = cnt + (cnt & 1)
        last_chunk = (off + cnt_p - 1) >> 7

        @pl.when(last_chunk >= gathered)
        def _():
            wait_gather(gathered & (_RING - 1))

        gathered = jnp.where(last_chunk >= gathered, gathered + 1, gathered)

        can_issue = (issued < nch) & ((issued - _RING) < (off >> 7))

        @pl.when(can_issue)
        def _():
            gather(issued)

        issued = jnp.where(can_issue, issued + 1, issued)

        half = srow >> 3

        @pl.when(((srow == 0) | (srow == _OB)) & (p >= 2 * _OB))
        def _():
            wait_flush(half)

        r0 = off & (_RROWS - 1)
        acc = [ring[r0, sl] for sl in sls]

        def fold(j, acc):
            rr = (off + 2 * j) & (_RROWS - 1)
            return tuple(
                jnp.maximum(jnp.maximum(a, ring[rr, sl]), ring[rr + 1, sl])
                for a, sl in zip(acc, sls))

        acc = tuple(jnp.maximum(a, ring[r0 + 1, sl])
                    for a, sl in zip(acc, sls))
        acc = lax.fori_loop(1, cnt_p >> 1, fold, acc)
        for a, sl in zip(acc, sls):
            stage[srow, sl] = a

        @pl.when((srow == _OB - 1) | (srow == 2 * _OB - 1))
        def _():
            s0 = pl.multiple_of((half << 3), _OB)
            d0 = pl.multiple_of(out_base + p - (_OB - 1), 8)
            pltpu.async_copy(stage.at[pl.ds(s0, _OB)],
                             out_hbm.at[pl.ds(d0, _OB)], sems_o.at[half])

        srow = jnp.where(srow == 2 * _OB - 1, 0, srow + 1)
        return off + cnt_p, gathered, issued, srow

    lax.fori_loop(0, pw, point,
                  (jnp.int32(0), jnp.int32(0), jnp.int32(_RING),
                   jnp.int32(0)))
    wait_flush(jnp.int32(0))
    wait_flush(jnp.int32(1))


_pool_call = functools.partial(
    pl.kernel,
    out_type=jax.ShapeDtypeStruct((_MP, _C), jnp.float32),
    mesh=plsc.VectorSubcoreMesh(core_axis_name="c", subcore_axis_name="s"),
    scratch_types=[
        pltpu.VMEM((_P * _K,), jnp.int32),
        pltpu.VMEM((_P + _L,), jnp.int32),
        pltpu.VMEM((_VMAX + _CH + _L,), jnp.int32),
        pltpu.VMEM((_RROWS, _C), jnp.float32),
        pltpu.VMEM((2 * _OB, _C), jnp.float32),
        pltpu.SemaphoreType.DMA((_RING,)),
        pltpu.SemaphoreType.DMA((2,)),
    ],
)(_pool_body)


def kernel(inputs, nn_count, nn_index):
    idx = nn_index.astype(jnp.int32)
    cnt = nn_count.astype(jnp.int32)
    return _pool_call(inputs, cnt, idx.reshape(-1))
